# baseline (speedup 1.0000x reference)
import functools

import jax
import jax.numpy as jnp
from jax import lax
from jax.experimental import pallas as pl
from jax.experimental.pallas import tpu as pltpu
from jax.experimental.pallas import tpu_sc as plsc

N = 10000
NP = 10240
E = 160000
CHUNK = 128
EP = 163840
NCHUNK = EP // CHUNK
NC, NS = 2, 16
RPT = NP // NS
DEGW = 128

_f32 = jnp.float32
_i32 = jnp.int32

_MESH = plsc.VectorSubcoreMesh(core_axis_name="c", subcore_axis_name="s")


@functools.partial(
    pl.kernel,
    out_type=jax.ShapeDtypeStruct((NC * NP, DEGW), _f32),
    mesh=_MESH,
    scratch_types=[
        pltpu.VMEM((NCHUNK // (NC * NS), CHUNK), _i32),
        pltpu.VMEM((CHUNK, DEGW), _f32),
        pltpu.VMEM((CHUNK, DEGW), _f32),
        pltpu.VMEM_SHARED((NP, DEGW), _f32),
        pltpu.SemaphoreType.DMA,
    ],
)
def _deg_kernel(dst_hbm, deg2_hbm, dstv, onesv, zerov, table, sem):
    c = lax.axis_index("c")
    s = lax.axis_index("s")
    w = s * NC + c
    npc = NCHUNK // (NC * NS)
    grp = 20
    pltpu.sync_copy(dst_hbm.at[pl.ds(w * npc, npc)], dstv)

    one16 = jnp.ones((16,), _f32)
    zero16 = jnp.zeros((16,), _f32)

    def fill(r, carry):
        for k in range(DEGW // 16):
            onesv[r, pl.ds(k * 16, 16)] = one16
            zerov[r, pl.ds(k * 16, 16)] = zero16
        return carry

    lax.fori_loop(0, CHUNK, fill, 0)
    for j in range(RPT // CHUNK):
        pltpu.sync_copy(zerov, table.at[pl.ds(s * RPT + j * CHUNK, CHUNK)])
    plsc.subcore_barrier()

    def body(j, carry):
        for b in range(grp):
            pltpu.async_copy(onesv, table.at[dstv.at[j * grp + b]], sem, add=True)
        for b in range(grp):
            pltpu.make_async_copy(onesv, table.at[dstv.at[j * grp + b]], sem).wait()
        return carry

    lax.fori_loop(0, npc // grp, body, 0)
    plsc.subcore_barrier()
    pltpu.sync_copy(table.at[pl.ds(s * RPT, RPT)],
                    deg2_hbm.at[pl.ds(c * NP + s * RPT, RPT)])


@functools.partial(
    pl.kernel,
    out_type=jax.ShapeDtypeStruct((NC * NP, 128), _f32),
    mesh=_MESH,
    scratch_types=[
        pltpu.VMEM((NCHUNK // NS // 2, CHUNK), _i32),
        pltpu.VMEM((NCHUNK // NS // 2, CHUNK), _i32),
        pltpu.VMEM((CHUNK, 128), _f32),
        pltpu.VMEM((CHUNK, 128), _f32),
        pltpu.VMEM_SHARED((NP, 128), _f32),
        pltpu.SemaphoreType.DMA,
        pltpu.SemaphoreType.DMA,
        pltpu.SemaphoreType.DMA,
        pltpu.SemaphoreType.DMA,
    ],
)
def _prop_kernel(g2_hbm, src_hbm, dst_hbm, acc2_hbm,
                 srcv, dstv, rows0, rows1, acc, semg0, semg1, sems0, sems1):
    c = lax.axis_index("c")
    s = lax.axis_index("s")
    npt = NCHUNK // NS
    nb = npt // 2
    g_half = g2_hbm.at[pl.ds(c * NP, NP)]
    pltpu.sync_copy(g_half.at[pl.ds(s * RPT, RPT)],
                    acc.at[pl.ds(s * RPT, RPT)])
    plsc.subcore_barrier()

    def gath(k, rows_x, semg_x):
        pltpu.async_copy(g_half.at[srcv.at[k]], rows_x, semg_x)

    def gath_wait(k, rows_x, semg_x):
        pltpu.make_async_copy(g_half.at[srcv.at[k]], rows_x, semg_x).wait()

    def scat(k, rows_x, sems_x):
        pltpu.async_copy(rows_x, acc.at[dstv.at[k]], sems_x, add=True)

    def scat_wait(k, rows_x, sems_x):
        pltpu.make_async_copy(rows_x, acc.at[dstv.at[k]], sems_x).wait()

    def step(k, rows_a, semg_a, sems_a, rows_b, semg_b, sems_b):
        @pl.when(k > 0)
        def _():
            scat_wait(k - 1, rows_b, sems_b)
        @pl.when(k + 1 < nb)
        def _():
            gath(k + 1, rows_b, semg_b)
        gath_wait(k, rows_a, semg_a)
        scat(k, rows_a, sems_a)

    def body(j, carry):
        step(2 * j, rows0, semg0, sems0, rows1, semg1, sems1)
        step(2 * j + 1, rows1, semg1, sems1, rows0, semg0, sems0)
        return carry

    for m in range(2):
        pltpu.sync_copy(src_hbm.at[pl.ds(s * npt + m * nb, nb)], srcv)
        pltpu.sync_copy(dst_hbm.at[pl.ds(s * npt + m * nb, nb)], dstv)
        gath(0, rows0, semg0)
        lax.fori_loop(0, nb // 2, body, 0)
        scat_wait(nb - 1, rows1, sems1)

    plsc.subcore_barrier()
    pltpu.sync_copy(acc.at[pl.ds(s * RPT, RPT)],
                    acc2_hbm.at[pl.ds(c * NP + s * RPT, RPT)])


@functools.partial(
    pl.kernel,
    out_type=jax.ShapeDtypeStruct((NC * NP, 128), _f32),
    mesh=_MESH,
    scratch_types=[
        pltpu.VMEM((NCHUNK // (NC * NS), CHUNK), _i32),
        pltpu.VMEM((NCHUNK // (NC * NS), CHUNK), _i32),
        pltpu.VMEM((CHUNK, 128), _f32),
        pltpu.VMEM((CHUNK, 128), _f32),
        pltpu.VMEM_SHARED((NP, 128), _f32),
        pltpu.SemaphoreType.DMA,
        pltpu.SemaphoreType.DMA,
        pltpu.SemaphoreType.DMA,
        pltpu.SemaphoreType.DMA,
    ],
)
def _prop3_kernel(g_hbm, src_hbm, dst_hbm, acc2_hbm,
                  srcv, dstv, rows0, rows1, acc, semg0, semg1, sems0, sems1):
    c = lax.axis_index("c")
    s = lax.axis_index("s")
    npc = NCHUNK // (NC * NS)
    base = c * (NCHUNK // NC) + s * npc
    pltpu.sync_copy(src_hbm.at[pl.ds(base, npc)], srcv)
    pltpu.sync_copy(dst_hbm.at[pl.ds(base, npc)], dstv)
    pltpu.sync_copy(g_hbm.at[pl.ds(s * RPT, RPT)], acc.at[pl.ds(s * RPT, RPT)])
    plsc.subcore_barrier()

    def gath(k, rows_x, semg_x):
        pltpu.async_copy(g_hbm.at[srcv.at[k]], rows_x, semg_x)

    def gath_wait(k, rows_x, semg_x):
        pltpu.make_async_copy(g_hbm.at[srcv.at[k]], rows_x, semg_x).wait()

    def scat(k, rows_x, sems_x):
        pltpu.async_copy(rows_x, acc.at[dstv.at[k]], sems_x, add=True)

    def scat_wait(k, rows_x, sems_x):
        pltpu.make_async_copy(rows_x, acc.at[dstv.at[k]], sems_x).wait()

    def step(k, rows_a, semg_a, sems_a, rows_b, semg_b, sems_b):
        @pl.when(k > 0)
        def _():
            scat_wait(k - 1, rows_b, sems_b)
        @pl.when(k + 1 < npc)
        def _():
            gath(k + 1, rows_b, semg_b)
        gath_wait(k, rows_a, semg_a)
        scat(k, rows_a, sems_a)

    def body(j, carry):
        step(2 * j, rows0, semg0, sems0, rows1, semg1, sems1)
        step(2 * j + 1, rows1, semg1, sems1, rows0, semg0, sems0)
        return carry

    gath(0, rows0, semg0)
    lax.fori_loop(0, npc // 2, body, 0)
    scat_wait(npc - 1, rows1, sems1)
    plsc.subcore_barrier()
    pltpu.sync_copy(acc.at[pl.ds(s * RPT, RPT)],
                    acc2_hbm.at[pl.ds(c * NP + s * RPT, RPT)])


_BLK = 5120
_GRID = NP // _BLK

_PREC = lax.Precision.DEFAULT


def _dot(a, b):
    return lax.dot_general(a, b, (((1,), (0,)), ((), ())),
                           preferred_element_type=_f32, precision=_PREC)


def _t0_body(x_ref, w1_ref, dega_ref, degb_ref, disb_ref, g_ref):
    deg = dega_ref[:, 0:1] + degb_ref[:, 0:1] + 1.0
    dis = lax.rsqrt(deg)
    disb_ref[...] = jnp.broadcast_to(dis, (_BLK, 128))
    u = (x_ref[...] * dis).astype(jnp.bfloat16)
    g = _dot(u, w1_ref[...])
    g_ref[0] = g[:, :128]
    g_ref[1] = g[:, 128:]


def _t1_body(acc_ref, disb_ref, b_ref, w_ref, g_ref):
    d = disb_ref[...]
    b = b_ref[...]
    zl = jnp.maximum(d * acc_ref[0] + b[:, :128], 0.0)
    zr = jnp.maximum(d * acc_ref[1] + b[:, 128:], 0.0)
    u = jnp.concatenate([d * zl, d * zr], axis=1).astype(jnp.bfloat16)
    g = _dot(u, w_ref[...])
    g_ref[0] = g[:, :128]
    g_ref[1] = g[:, 128:]


def _t2_body(acc_ref, disb_ref, b_ref, w_ref, g3_ref):
    d = disb_ref[...]
    b = b_ref[...]
    zl = jnp.maximum(d * acc_ref[0] + b[:, :128], 0.0)
    zr = jnp.maximum(d * acc_ref[1] + b[:, 128:], 0.0)
    u = jnp.concatenate([d * zl, d * zr], axis=1).astype(jnp.bfloat16)
    g3_ref[...] = _dot(u, w_ref[...])


def _t3_body(acc_ref, g3_ref, disb_ref, b_ref, out_ref):
    out_ref[...] = (disb_ref[...] * (acc_ref[0] + acc_ref[1] - g3_ref[...])
                    + b_ref[...])


def _row_spec(width):
    return pl.BlockSpec((_BLK, width), lambda i: (i, 0))


def _halves_spec():
    return pl.BlockSpec((2, _BLK, 128), lambda i: (0, i, 0))


def _full_spec(shape):
    return pl.BlockSpec(shape, lambda i: (0,) * len(shape))


def _t0_call(x, W1, deg2):
    return pl.pallas_call(
        _t0_body,
        grid=(_GRID,),
        in_specs=[_row_spec(256), _full_spec((256, 256)),
                  pl.BlockSpec((_BLK, DEGW), lambda i: (i, 0)),
                  pl.BlockSpec((_BLK, DEGW), lambda i: (i + NP // _BLK, 0))],
        out_specs=[_row_spec(128), _halves_spec()],
        out_shape=[jax.ShapeDtypeStruct((NP, 128), _f32),
                   jax.ShapeDtypeStruct((2, NP, 128), _f32)],
    )(x, W1, deg2, deg2)


def _t1_call(acc2, disb, b, W):
    return pl.pallas_call(
        _t1_body,
        grid=(_GRID,),
        in_specs=[_halves_spec(), _row_spec(128),
                  _full_spec((1, 256)), _full_spec((256, 256))],
        out_specs=[_halves_spec()],
        out_shape=[jax.ShapeDtypeStruct((2, NP, 128), _f32)],
    )(acc2, disb, b, W)[0]


def _t2_call(acc2, disb, b, W):
    return pl.pallas_call(
        _t2_body,
        grid=(_GRID,),
        in_specs=[_halves_spec(), _row_spec(128),
                  _full_spec((1, 256)), _full_spec((256, 128))],
        out_specs=[_row_spec(128)],
        out_shape=[jax.ShapeDtypeStruct((NP, 128), _f32)],
    )(acc2, disb, b, W)[0]


def _t3_call(acc2, g3, disb, b):
    return pl.pallas_call(
        _t3_body,
        grid=(_GRID,),
        in_specs=[_halves_spec(), _row_spec(128), _row_spec(128),
                  _full_spec((1, 128))],
        out_specs=[_row_spec(128)],
        out_shape=[jax.ShapeDtypeStruct((N, 128), _f32)],
    )(acc2, g3, disb, b)[0]


def kernel(x, edge_index, W1, b1, W2, b2, W3, b3):
    src = edge_index[0].astype(_i32)
    dst = edge_index[1].astype(_i32)
    pad = EP - E
    api = jnp.arange(pad, dtype=_i32)
    src_p = jnp.concatenate([src, api % N])
    dst_p = jnp.concatenate([dst, N + api % (NP - N)])
    src2 = src_p.reshape(NCHUNK, CHUNK)
    dst2 = dst_p.reshape(NCHUNK, CHUNK)

    deg2 = _deg_kernel(dst2)

    disb, g1 = _t0_call(x, W1.astype(jnp.bfloat16), deg2)
    a1 = _prop_kernel(g1.reshape(2 * NP, 128), src2, dst2)
    W2b = W2.astype(jnp.bfloat16)
    g2 = _t1_call(a1.reshape(2, NP, 128), disb, b1.reshape(1, 256), W2b)
    a2 = _prop_kernel(g2.reshape(2 * NP, 128), src2, dst2)
    g3 = _t2_call(a2.reshape(2, NP, 128), disb, b2.reshape(1, 256), W3.astype(jnp.bfloat16))
    a3 = _prop3_kernel(g3, src2, dst2)
    return _t3_call(a3.reshape(2, NP, 128), g3, disb, b3.reshape(1, 128))


# speedup vs baseline: 1.0025x; 1.0025x over previous
import functools

import jax
import jax.numpy as jnp
from jax import lax
from jax.experimental import pallas as pl
from jax.experimental.pallas import tpu as pltpu
from jax.experimental.pallas import tpu_sc as plsc

N = 10000
NP = 10240
E = 160000
CHUNK = 128
EP = 163840
NCHUNK = EP // CHUNK
NC, NS = 2, 16
RPT = NP // NS
DEGW = 128

_f32 = jnp.float32
_i32 = jnp.int32

_MESH = plsc.VectorSubcoreMesh(core_axis_name="c", subcore_axis_name="s")


@functools.partial(
    pl.kernel,
    out_type=jax.ShapeDtypeStruct((NC * NP, DEGW), _f32),
    mesh=_MESH,
    scratch_types=[
        pltpu.VMEM((NCHUNK // (NC * NS), CHUNK), _i32),
        pltpu.VMEM((CHUNK, DEGW), _f32),
        pltpu.VMEM((CHUNK, DEGW), _f32),
        pltpu.VMEM_SHARED((NP, DEGW), _f32),
        pltpu.SemaphoreType.DMA,
    ],
)
def _deg_kernel(dst_hbm, deg2_hbm, dstv, onesv, zerov, table, sem):
    c = lax.axis_index("c")
    s = lax.axis_index("s")
    w = s * NC + c
    npc = NCHUNK // (NC * NS)
    grp = 20
    pltpu.sync_copy(dst_hbm.at[pl.ds(w * npc, npc)], dstv)

    one16 = jnp.ones((16,), _f32)
    zero16 = jnp.zeros((16,), _f32)

    def fill(r, carry):
        for k in range(DEGW // 16):
            onesv[r, pl.ds(k * 16, 16)] = one16
            zerov[r, pl.ds(k * 16, 16)] = zero16
        return carry

    lax.fori_loop(0, CHUNK, fill, 0)
    for j in range(RPT // CHUNK):
        pltpu.sync_copy(zerov, table.at[pl.ds(s * RPT + j * CHUNK, CHUNK)])
    plsc.subcore_barrier()

    def body(j, carry):
        for b in range(grp):
            pltpu.async_copy(onesv, table.at[dstv.at[j * grp + b]], sem, add=True)
        for b in range(grp):
            pltpu.make_async_copy(onesv, table.at[dstv.at[j * grp + b]], sem).wait()
        return carry

    lax.fori_loop(0, npc // grp, body, 0)
    plsc.subcore_barrier()
    pltpu.sync_copy(table.at[pl.ds(s * RPT, RPT)],
                    deg2_hbm.at[pl.ds(c * NP + s * RPT, RPT)])


@functools.partial(
    pl.kernel,
    out_type=jax.ShapeDtypeStruct((NC * NP, 128), _f32),
    mesh=_MESH,
    scratch_types=[
        pltpu.VMEM((NCHUNK // NS // 2, CHUNK), _i32),
        pltpu.VMEM((NCHUNK // NS // 2, CHUNK), _i32),
        pltpu.VMEM((CHUNK, 128), _f32),
        pltpu.VMEM((CHUNK, 128), _f32),
        pltpu.VMEM_SHARED((NP, 128), _f32),
        pltpu.SemaphoreType.DMA,
        pltpu.SemaphoreType.DMA,
        pltpu.SemaphoreType.DMA,
        pltpu.SemaphoreType.DMA,
    ],
)
def _prop_kernel(g2_hbm, src_hbm, dst_hbm, acc2_hbm,
                 srcv, dstv, rows0, rows1, acc, semg0, semg1, sems0, sems1):
    c = lax.axis_index("c")
    s = lax.axis_index("s")
    npt = NCHUNK // NS
    nb = npt // 2
    g_half = g2_hbm.at[pl.ds(c * NP, NP)]

    zero16 = jnp.zeros((16,), _f32)

    def zfill(r, carry):
        for k in range(8):
            rows0[r, pl.ds(k * 16, 16)] = zero16
        return carry

    lax.fori_loop(0, CHUNK, zfill, 0)
    for j in range(RPT // CHUNK):
        pltpu.sync_copy(rows0, acc.at[pl.ds(s * RPT + j * CHUNK, CHUNK)])
    plsc.subcore_barrier()

    def gath(k, rows_x, semg_x):
        pltpu.async_copy(g_half.at[srcv.at[k]], rows_x, semg_x)

    def gath_wait(k, rows_x, semg_x):
        pltpu.make_async_copy(g_half.at[srcv.at[k]], rows_x, semg_x).wait()

    def scat(k, rows_x, sems_x):
        pltpu.async_copy(rows_x, acc.at[dstv.at[k]], sems_x, add=True)

    def scat_wait(k, rows_x, sems_x):
        pltpu.make_async_copy(rows_x, acc.at[dstv.at[k]], sems_x).wait()

    def step(k, rows_a, semg_a, sems_a, rows_b, semg_b, sems_b):
        @pl.when(k > 0)
        def _():
            scat_wait(k - 1, rows_b, sems_b)
        @pl.when(k + 1 < nb)
        def _():
            gath(k + 1, rows_b, semg_b)
        gath_wait(k, rows_a, semg_a)
        scat(k, rows_a, sems_a)

    def body(j, carry):
        step(2 * j, rows0, semg0, sems0, rows1, semg1, sems1)
        step(2 * j + 1, rows1, semg1, sems1, rows0, semg0, sems0)
        return carry

    for m in range(2):
        pltpu.sync_copy(src_hbm.at[pl.ds(s * npt + m * nb, nb)], srcv)
        pltpu.sync_copy(dst_hbm.at[pl.ds(s * npt + m * nb, nb)], dstv)
        gath(0, rows0, semg0)
        lax.fori_loop(0, nb // 2, body, 0)
        scat_wait(nb - 1, rows1, sems1)

    plsc.subcore_barrier()
    pltpu.sync_copy(acc.at[pl.ds(s * RPT, RPT)],
                    acc2_hbm.at[pl.ds(c * NP + s * RPT, RPT)])


@functools.partial(
    pl.kernel,
    out_type=jax.ShapeDtypeStruct((NC * NP, 128), _f32),
    mesh=_MESH,
    scratch_types=[
        pltpu.VMEM((NCHUNK // (NC * NS), CHUNK), _i32),
        pltpu.VMEM((NCHUNK // (NC * NS), CHUNK), _i32),
        pltpu.VMEM((CHUNK, 128), _f32),
        pltpu.VMEM((CHUNK, 128), _f32),
        pltpu.VMEM_SHARED((NP, 128), _f32),
        pltpu.SemaphoreType.DMA,
        pltpu.SemaphoreType.DMA,
        pltpu.SemaphoreType.DMA,
        pltpu.SemaphoreType.DMA,
    ],
)
def _prop3_kernel(g_hbm, src_hbm, dst_hbm, acc2_hbm,
                  srcv, dstv, rows0, rows1, acc, semg0, semg1, sems0, sems1):
    c = lax.axis_index("c")
    s = lax.axis_index("s")
    npc = NCHUNK // (NC * NS)
    base = c * (NCHUNK // NC) + s * npc
    pltpu.sync_copy(src_hbm.at[pl.ds(base, npc)], srcv)
    pltpu.sync_copy(dst_hbm.at[pl.ds(base, npc)], dstv)
    zero16 = jnp.zeros((16,), _f32)

    def zfill(r, carry):
        for k in range(8):
            rows0[r, pl.ds(k * 16, 16)] = zero16
        return carry

    lax.fori_loop(0, CHUNK, zfill, 0)
    for j in range(RPT // CHUNK):
        pltpu.sync_copy(rows0, acc.at[pl.ds(s * RPT + j * CHUNK, CHUNK)])
    plsc.subcore_barrier()

    def gath(k, rows_x, semg_x):
        pltpu.async_copy(g_hbm.at[srcv.at[k]], rows_x, semg_x)

    def gath_wait(k, rows_x, semg_x):
        pltpu.make_async_copy(g_hbm.at[srcv.at[k]], rows_x, semg_x).wait()

    def scat(k, rows_x, sems_x):
        pltpu.async_copy(rows_x, acc.at[dstv.at[k]], sems_x, add=True)

    def scat_wait(k, rows_x, sems_x):
        pltpu.make_async_copy(rows_x, acc.at[dstv.at[k]], sems_x).wait()

    def step(k, rows_a, semg_a, sems_a, rows_b, semg_b, sems_b):
        @pl.when(k > 0)
        def _():
            scat_wait(k - 1, rows_b, sems_b)
        @pl.when(k + 1 < npc)
        def _():
            gath(k + 1, rows_b, semg_b)
        gath_wait(k, rows_a, semg_a)
        scat(k, rows_a, sems_a)

    def body(j, carry):
        step(2 * j, rows0, semg0, sems0, rows1, semg1, sems1)
        step(2 * j + 1, rows1, semg1, sems1, rows0, semg0, sems0)
        return carry

    gath(0, rows0, semg0)
    lax.fori_loop(0, npc // 2, body, 0)
    scat_wait(npc - 1, rows1, sems1)
    plsc.subcore_barrier()
    pltpu.sync_copy(acc.at[pl.ds(s * RPT, RPT)],
                    acc2_hbm.at[pl.ds(c * NP + s * RPT, RPT)])


_BLK = 5120
_GRID = NP // _BLK

_PREC = lax.Precision.DEFAULT


def _dot(a, b):
    return lax.dot_general(a, b, (((1,), (0,)), ((), ())),
                           preferred_element_type=_f32, precision=_PREC)


def _t0_body(x_ref, w1_ref, dega_ref, degb_ref, disb_ref, g_ref):
    deg = dega_ref[:, 0:1] + degb_ref[:, 0:1] + 1.0
    dis = lax.rsqrt(deg)
    disb_ref[...] = jnp.broadcast_to(dis, (_BLK, 128))
    u = (x_ref[...] * dis).astype(jnp.bfloat16)
    g = _dot(u, w1_ref[...])
    g_ref[0] = g[:, :128]
    g_ref[1] = g[:, 128:]


def _t1_body(acc_ref, gin_ref, disb_ref, b_ref, w_ref, g_ref):
    d = disb_ref[...]
    b = b_ref[...]
    zl = jnp.maximum(d * (acc_ref[0] + gin_ref[0]) + b[:, :128], 0.0)
    zr = jnp.maximum(d * (acc_ref[1] + gin_ref[1]) + b[:, 128:], 0.0)
    u = jnp.concatenate([d * zl, d * zr], axis=1).astype(jnp.bfloat16)
    g = _dot(u, w_ref[...])
    g_ref[0] = g[:, :128]
    g_ref[1] = g[:, 128:]


def _t2_body(acc_ref, gin_ref, disb_ref, b_ref, w_ref, g3_ref):
    d = disb_ref[...]
    b = b_ref[...]
    zl = jnp.maximum(d * (acc_ref[0] + gin_ref[0]) + b[:, :128], 0.0)
    zr = jnp.maximum(d * (acc_ref[1] + gin_ref[1]) + b[:, 128:], 0.0)
    u = jnp.concatenate([d * zl, d * zr], axis=1).astype(jnp.bfloat16)
    g3_ref[...] = _dot(u, w_ref[...])


def _t3_body(acc_ref, g3_ref, disb_ref, b_ref, out_ref):
    out_ref[...] = (disb_ref[...] * (acc_ref[0] + acc_ref[1] + g3_ref[...])
                    + b_ref[...])


def _row_spec(width):
    return pl.BlockSpec((_BLK, width), lambda i: (i, 0))


def _halves_spec():
    return pl.BlockSpec((2, _BLK, 128), lambda i: (0, i, 0))


def _full_spec(shape):
    return pl.BlockSpec(shape, lambda i: (0,) * len(shape))


def _t0_call(x, W1, deg2):
    return pl.pallas_call(
        _t0_body,
        grid=(_GRID,),
        in_specs=[_row_spec(256), _full_spec((256, 256)),
                  pl.BlockSpec((_BLK, DEGW), lambda i: (i, 0)),
                  pl.BlockSpec((_BLK, DEGW), lambda i: (i + NP // _BLK, 0))],
        out_specs=[_row_spec(128), _halves_spec()],
        out_shape=[jax.ShapeDtypeStruct((NP, 128), _f32),
                   jax.ShapeDtypeStruct((2, NP, 128), _f32)],
    )(x, W1, deg2, deg2)


def _t1_call(acc2, gin, disb, b, W):
    return pl.pallas_call(
        _t1_body,
        grid=(_GRID,),
        in_specs=[_halves_spec(), _halves_spec(), _row_spec(128),
                  _full_spec((1, 256)), _full_spec((256, 256))],
        out_specs=[_halves_spec()],
        out_shape=[jax.ShapeDtypeStruct((2, NP, 128), _f32)],
    )(acc2, gin, disb, b, W)[0]


def _t2_call(acc2, gin, disb, b, W):
    return pl.pallas_call(
        _t2_body,
        grid=(_GRID,),
        in_specs=[_halves_spec(), _halves_spec(), _row_spec(128),
                  _full_spec((1, 256)), _full_spec((256, 128))],
        out_specs=[_row_spec(128)],
        out_shape=[jax.ShapeDtypeStruct((NP, 128), _f32)],
    )(acc2, gin, disb, b, W)[0]


def _t3_call(acc2, g3, disb, b):
    return pl.pallas_call(
        _t3_body,
        grid=(_GRID,),
        in_specs=[_halves_spec(), _row_spec(128), _row_spec(128),
                  _full_spec((1, 128))],
        out_specs=[_row_spec(128)],
        out_shape=[jax.ShapeDtypeStruct((N, 128), _f32)],
    )(acc2, g3, disb, b)[0]


def kernel(x, edge_index, W1, b1, W2, b2, W3, b3):
    src = edge_index[0].astype(_i32)
    dst = edge_index[1].astype(_i32)
    pad = EP - E
    api = jnp.arange(pad, dtype=_i32)
    src_p = jnp.concatenate([src, api % N])
    dst_p = jnp.concatenate([dst, N + api % (NP - N)])
    src2 = src_p.reshape(NCHUNK, CHUNK)
    dst2 = dst_p.reshape(NCHUNK, CHUNK)

    deg2 = _deg_kernel(dst2)

    disb, g1 = _t0_call(x, W1.astype(jnp.bfloat16), deg2)
    a1 = _prop_kernel(g1.reshape(2 * NP, 128), src2, dst2)
    g2 = _t1_call(a1.reshape(2, NP, 128), g1, disb, b1.reshape(1, 256), W2.astype(jnp.bfloat16))
    a2 = _prop_kernel(g2.reshape(2 * NP, 128), src2, dst2)
    g3 = _t2_call(a2.reshape(2, NP, 128), g2, disb, b2.reshape(1, 256), W3.astype(jnp.bfloat16))
    a3 = _prop3_kernel(g3, src2, dst2)
    return _t3_call(a3.reshape(2, NP, 128), g3, disb, b3.reshape(1, 128))
